# Initial kernel scaffold; baseline (speedup 1.0000x reference)
import jax
import jax.numpy as jnp
from jax.experimental import pallas as pl


def kernel(x, edge_index, batch, W1, b1, W2, b2, Wo, bo):
    raise NotImplementedError("write your pallas kernel here")


# speedup vs baseline: 18.8007x; 18.8007x over previous
import functools

import jax
import jax.numpy as jnp
from jax import lax
from jax.experimental import pallas as pl
from jax.experimental.pallas import tpu as pltpu
from jax.experimental.pallas import tpu_sc as plsc

N = 10000
E = 320000
D = 128
H = 128
G = 64
C_OUT = 19

NC = 2
NS = 16
NW = NC * NS
EPT = E // NW
K = 80
CH = EPT // K
RPT = N // NS
RCH = 125
BN = 1000
NBLK = N // BN


def _sc_scatter_body(src_hbm, dst_hbm, hs_hbm, zeros_hbm, out_hbm,
                     idx_s, idx_d, rows, zbuf, acc, sem):
    c = lax.axis_index("c")
    s = lax.axis_index("s")
    wid = c * NS + s
    pltpu.sync_copy(src_hbm.at[wid], idx_s)
    pltpu.sync_copy(dst_hbm.at[wid], idx_d)
    pltpu.sync_copy(zeros_hbm, zbuf)
    row0 = s * RPT
    for r in range(RPT // RCH):
        pltpu.sync_copy(zbuf, acc.at[pl.ds(row0 + r * RCH, RCH)])
    plsc.subcore_barrier()

    def body(i, carry):
        pltpu.async_copy(hs_hbm.at[idx_s.at[i]], rows, sem).wait()
        pltpu.sync_copy(rows, acc.at[idx_d.at[i]], add=True)
        return carry

    lax.fori_loop(0, CH, body, 0)
    plsc.subcore_barrier()
    w0 = s * 624
    pltpu.sync_copy(acc.at[pl.ds(w0, 624)], out_hbm.at[c, pl.ds(w0, 624)])

    @pl.when(s == NS - 1)
    def _():
        pltpu.sync_copy(acc.at[pl.ds(9984, 16)],
                        out_hbm.at[c, pl.ds(9984, 16)])


_sc_scatter = functools.partial(
    pl.kernel,
    out_type=jax.ShapeDtypeStruct((NC, N, H), jnp.float32),
    mesh=plsc.VectorSubcoreMesh(core_axis_name="c", subcore_axis_name="s"),
    scratch_types=[
        pltpu.VMEM((CH, K), jnp.int32),
        pltpu.VMEM((CH, K), jnp.int32),
        pltpu.VMEM((K, H), jnp.float32),
        pltpu.VMEM((RCH, H), jnp.float32),
        pltpu.VMEM_SHARED((N, H), jnp.float32),
        pltpu.SemaphoreType.DMA,
    ],
    compiler_params=pltpu.CompilerParams(use_tc_tiling_on_sc=False),
)(_sc_scatter_body)


def _sc_deg_body(dst_hbm, ones_hbm, zeros_hbm, out_hbm,
                 idx_d, onesb, z16, acc, sem):
    c = lax.axis_index("c")
    s = lax.axis_index("s")
    wid = c * NS + s
    pltpu.sync_copy(dst_hbm.at[wid], idx_d)
    pltpu.sync_copy(ones_hbm, onesb)
    pltpu.sync_copy(zeros_hbm, z16)
    row0 = s * RPT
    pltpu.sync_copy(z16, acc.at[pl.ds(row0, RPT)])
    plsc.subcore_barrier()

    def body(i, carry):
        pltpu.sync_copy(onesb, acc.at[idx_d.at[i]], add=True)
        return carry

    lax.fori_loop(0, CH, body, 0)
    plsc.subcore_barrier()
    w0 = s * 624
    pltpu.sync_copy(acc.at[pl.ds(w0, 624)], out_hbm.at[c, pl.ds(w0, 624)])

    @pl.when(s == NS - 1)
    def _():
        pltpu.sync_copy(acc.at[pl.ds(9984, 16)],
                        out_hbm.at[c, pl.ds(9984, 16)])


_sc_deg = functools.partial(
    pl.kernel,
    out_type=jax.ShapeDtypeStruct((NC, N, 16), jnp.float32),
    mesh=plsc.VectorSubcoreMesh(core_axis_name="c", subcore_axis_name="s"),
    scratch_types=[
        pltpu.VMEM((CH, K), jnp.int32),
        pltpu.VMEM((K, 16), jnp.float32),
        pltpu.VMEM((RPT, 16), jnp.float32),
        pltpu.VMEM_SHARED((N, 16), jnp.float32),
        pltpu.SemaphoreType.DMA,
    ],
    compiler_params=pltpu.CompilerParams(use_tc_tiling_on_sc=False),
)(_sc_deg_body)


def _tc_b_body(deg0_ref, deg1_ref, x_ref, w1_ref, dis_ref, hs_ref):
    deg = deg0_ref[...] + deg1_ref[...] + 1.0
    dis = lax.rsqrt(deg)
    dis_ref[...] = dis
    xw = jnp.dot(x_ref[...], w1_ref[...], preferred_element_type=jnp.float32)
    hs_ref[...] = xw * dis[:, 0:1]


_tc_b = pl.pallas_call(
    _tc_b_body,
    grid=(NBLK,),
    in_specs=[
        pl.BlockSpec((BN, 16), lambda i: (i, 0)),
        pl.BlockSpec((BN, 16), lambda i: (i, 0)),
        pl.BlockSpec((BN, D), lambda i: (i, 0)),
        pl.BlockSpec((D, H), lambda i: (0, 0)),
    ],
    out_specs=[
        pl.BlockSpec((BN, 16), lambda i: (i, 0)),
        pl.BlockSpec((BN, H), lambda i: (i, 0)),
    ],
    out_shape=[
        jax.ShapeDtypeStruct((N, 16), jnp.float32),
        jax.ShapeDtypeStruct((N, H), jnp.float32),
    ],
)


def _tc_d_body(p0_ref, p1_ref, hs1_ref, dis_ref, b1_ref, w2_ref, hs2_ref):
    dis = dis_ref[...][:, 0:1]
    acc = p0_ref[...] + p1_ref[...] + hs1_ref[...]
    h1 = jnp.maximum(acc * dis + b1_ref[...], 0.0)
    hw = jnp.dot(h1, w2_ref[...], preferred_element_type=jnp.float32)
    hs2_ref[...] = hw * dis


_tc_d = pl.pallas_call(
    _tc_d_body,
    grid=(NBLK,),
    in_specs=[
        pl.BlockSpec((BN, H), lambda i: (i, 0)),
        pl.BlockSpec((BN, H), lambda i: (i, 0)),
        pl.BlockSpec((BN, H), lambda i: (i, 0)),
        pl.BlockSpec((BN, 16), lambda i: (i, 0)),
        pl.BlockSpec((1, H), lambda i: (0, 0)),
        pl.BlockSpec((H, H), lambda i: (0, 0)),
    ],
    out_specs=pl.BlockSpec((BN, H), lambda i: (i, 0)),
    out_shape=jax.ShapeDtypeStruct((N, H), jnp.float32),
)


def _tc_f_body(q0_ref, q1_ref, hs2_ref, dis_ref, b2_ref, batch_ref,
               wo_ref, bo_ref, out_ref, sum_ref, cnt_ref):
    g = pl.program_id(0)
    dis = dis_ref[...][:, 0:1]
    h2 = (q0_ref[...] + q1_ref[...] + hs2_ref[...]) * dis + b2_ref[...]
    bblk = batch_ref[0]
    gids = lax.broadcasted_iota(jnp.int32, (G, BN), 0)
    oh = (gids == bblk).astype(jnp.float32)
    psum = jnp.dot(oh, h2, preferred_element_type=jnp.float32)
    pcnt = jnp.broadcast_to(jnp.sum(oh, axis=1, keepdims=True), (G, H))

    @pl.when(g == 0)
    def _():
        sum_ref[...] = jnp.zeros_like(sum_ref)
        cnt_ref[...] = jnp.zeros_like(cnt_ref)

    sum_ref[...] += psum
    cnt_ref[...] += pcnt

    @pl.when(g == NBLK - 1)
    def _():
        pooled = sum_ref[...] / jnp.maximum(cnt_ref[...], 1.0)
        out_ref[...] = (
            jnp.dot(pooled, wo_ref[...], preferred_element_type=jnp.float32)
            + bo_ref[...]
        )


_tc_f = pl.pallas_call(
    _tc_f_body,
    grid=(NBLK,),
    in_specs=[
        pl.BlockSpec((BN, H), lambda i: (i, 0)),
        pl.BlockSpec((BN, H), lambda i: (i, 0)),
        pl.BlockSpec((BN, H), lambda i: (i, 0)),
        pl.BlockSpec((BN, 16), lambda i: (i, 0)),
        pl.BlockSpec((1, H), lambda i: (0, 0)),
        pl.BlockSpec((1, 1, BN), lambda i: (i, 0, 0)),
        pl.BlockSpec((H, 128), lambda i: (0, 0)),
        pl.BlockSpec((1, 128), lambda i: (0, 0)),
    ],
    out_specs=pl.BlockSpec((G, 128), lambda i: (0, 0)),
    out_shape=jax.ShapeDtypeStruct((G, 128), jnp.float32),
    scratch_shapes=[
        pltpu.VMEM((G, H), jnp.float32),
        pltpu.VMEM((G, H), jnp.float32),
    ],
)


def kernel(x, edge_index, batch, W1, b1, W2, b2, Wo, bo):
    src = edge_index[0].reshape(NW, CH, K)
    dst = edge_index[1].reshape(NW, CH, K)
    zeros_h = jnp.zeros((RCH, H), jnp.float32)
    zeros16 = jnp.zeros((RPT, 16), jnp.float32)
    ones16 = jnp.ones((K, 16), jnp.float32)

    degp = _sc_deg(dst, ones16, zeros16)
    dis16, hs1 = _tc_b(degp[0], degp[1], x, W1)
    p = _sc_scatter(src, dst, hs1, zeros_h)
    hs2 = _tc_d(p[0], p[1], hs1, dis16, b1.reshape(1, H), W2)
    q = _sc_scatter(src, dst, hs2, zeros_h)
    wo_pad = jnp.pad(Wo, ((0, 0), (0, 128 - C_OUT)))
    bo_pad = jnp.pad(bo, (0, 128 - C_OUT)).reshape(1, 128)
    outp = _tc_f(q[0], q[1], hs2, dis16, b2.reshape(1, H),
                 batch.reshape(NBLK, 1, BN), wo_pad, bo_pad)
    return outp[:, :C_OUT]


# speedup vs baseline: 27.8680x; 1.4823x over previous
import functools

import jax
import jax.numpy as jnp
from jax import lax
from jax.experimental import pallas as pl
from jax.experimental.pallas import tpu as pltpu
from jax.experimental.pallas import tpu_sc as plsc

N = 10000
E = 320000
D = 128
H = 128
G = 64
C_OUT = 19

NC = 2
NS = 16
NW = NC * NS
EPT = E // NW
K = 80
CH = EPT // K
RPT = N // NS
RCH = 125
BN = 1000
NBLK = N // BN


def _sc_scatter_body(src_hbm, dst_hbm, hs_hbm, zeros_hbm, out_hbm,
                     idx_s, idx_d, rows_a, rows_b, acc, sem_a, sem_b):
    c = lax.axis_index("c")
    s = lax.axis_index("s")
    wid = c * NS + s
    pltpu.sync_copy(src_hbm.at[wid], idx_s)
    pltpu.sync_copy(dst_hbm.at[wid], idx_d)
    row0 = s * RPT
    pltpu.sync_copy(zeros_hbm, acc.at[pl.ds(row0, RPT)])
    plsc.subcore_barrier()

    pltpu.async_copy(hs_hbm.at[idx_s.at[0]], rows_a, sem_a)

    def body(j, carry):
        i0 = 2 * j
        pltpu.async_copy(hs_hbm.at[idx_s.at[i0 + 1]], rows_b, sem_b)
        pltpu.make_async_copy(hs_hbm.at[idx_s.at[i0]], rows_a, sem_a).wait()
        pltpu.sync_copy(rows_a, acc.at[idx_d.at[i0]], add=True)

        @pl.when(j < CH // 2 - 1)
        def _():
            pltpu.async_copy(hs_hbm.at[idx_s.at[i0 + 2]], rows_a, sem_a)

        pltpu.make_async_copy(hs_hbm.at[idx_s.at[i0 + 1]], rows_b,
                              sem_b).wait()
        pltpu.sync_copy(rows_b, acc.at[idx_d.at[i0 + 1]], add=True)
        return carry

    lax.fori_loop(0, CH // 2, body, 0)
    if CH % 2 == 1:
        pltpu.async_copy(hs_hbm.at[idx_s.at[CH - 1]], rows_a, sem_a).wait()
        pltpu.sync_copy(rows_a, acc.at[idx_d.at[CH - 1]], add=True)
    plsc.subcore_barrier()
    w0 = s * 624
    pltpu.sync_copy(acc.at[pl.ds(w0, 624)], out_hbm.at[c, pl.ds(w0, 624)])

    @pl.when(s == NS - 1)
    def _():
        pltpu.sync_copy(acc.at[pl.ds(9984, 16)],
                        out_hbm.at[c, pl.ds(9984, 16)])


_sc_scatter = functools.partial(
    pl.kernel,
    out_type=jax.ShapeDtypeStruct((NC, N, H), jnp.float32),
    mesh=plsc.VectorSubcoreMesh(core_axis_name="c", subcore_axis_name="s"),
    scratch_types=[
        pltpu.VMEM((CH, K), jnp.int32),
        pltpu.VMEM((CH, K), jnp.int32),
        pltpu.VMEM((K, H), jnp.float32),
        pltpu.VMEM((K, H), jnp.float32),
        pltpu.VMEM_SHARED((N, H), jnp.float32),
        pltpu.SemaphoreType.DMA,
        pltpu.SemaphoreType.DMA,
    ],
    compiler_params=pltpu.CompilerParams(use_tc_tiling_on_sc=False),
)(_sc_scatter_body)


def _sc_deg_body(dst_hbm, ones_hbm, zeros_hbm, out_hbm,
                 idx_d, onesb, acc, sem):
    c = lax.axis_index("c")
    s = lax.axis_index("s")
    wid = c * NS + s
    pltpu.sync_copy(dst_hbm.at[wid], idx_d)
    pltpu.sync_copy(ones_hbm, onesb)
    row0 = s * RPT
    pltpu.sync_copy(zeros_hbm, acc.at[pl.ds(row0, RPT)])
    plsc.subcore_barrier()

    def body(i, carry):
        pltpu.sync_copy(onesb, acc.at[idx_d.at[i]], add=True)
        return carry

    lax.fori_loop(0, CH, body, 0)
    plsc.subcore_barrier()
    w0 = s * 624
    pltpu.sync_copy(acc.at[pl.ds(w0, 624)], out_hbm.at[c, pl.ds(w0, 624)])

    @pl.when(s == NS - 1)
    def _():
        pltpu.sync_copy(acc.at[pl.ds(9984, 16)],
                        out_hbm.at[c, pl.ds(9984, 16)])


_sc_deg = functools.partial(
    pl.kernel,
    out_type=jax.ShapeDtypeStruct((NC, N, 16), jnp.float32),
    mesh=plsc.VectorSubcoreMesh(core_axis_name="c", subcore_axis_name="s"),
    scratch_types=[
        pltpu.VMEM((CH, K), jnp.int32),
        pltpu.VMEM((K, 16), jnp.float32),
        pltpu.VMEM_SHARED((N, 16), jnp.float32),
        pltpu.SemaphoreType.DMA,
    ],
    compiler_params=pltpu.CompilerParams(use_tc_tiling_on_sc=False),
)(_sc_deg_body)


def _tc_b_body(deg0_ref, deg1_ref, x_ref, w1_ref, dis_ref, hs_ref):
    deg = deg0_ref[...] + deg1_ref[...] + 1.0
    dis = lax.rsqrt(deg)
    dis_ref[...] = dis
    xw = jnp.dot(x_ref[...], w1_ref[...], preferred_element_type=jnp.float32)
    hs_ref[...] = xw * dis[:, 0:1]


_tc_b = pl.pallas_call(
    _tc_b_body,
    grid=(NBLK,),
    in_specs=[
        pl.BlockSpec((BN, 16), lambda i: (i, 0)),
        pl.BlockSpec((BN, 16), lambda i: (i, 0)),
        pl.BlockSpec((BN, D), lambda i: (i, 0)),
        pl.BlockSpec((D, H), lambda i: (0, 0)),
    ],
    out_specs=[
        pl.BlockSpec((BN, 16), lambda i: (i, 0)),
        pl.BlockSpec((BN, H), lambda i: (i, 0)),
    ],
    out_shape=[
        jax.ShapeDtypeStruct((N, 16), jnp.float32),
        jax.ShapeDtypeStruct((N, H), jnp.float32),
    ],
)


def _tc_d_body(p0_ref, p1_ref, hs1_ref, dis_ref, b1_ref, w2_ref, hs2_ref):
    dis = dis_ref[...][:, 0:1]
    acc = p0_ref[...] + p1_ref[...] + hs1_ref[...]
    h1 = jnp.maximum(acc * dis + b1_ref[...], 0.0)
    hw = jnp.dot(h1, w2_ref[...], preferred_element_type=jnp.float32)
    hs2_ref[...] = hw * dis


_tc_d = pl.pallas_call(
    _tc_d_body,
    grid=(NBLK,),
    in_specs=[
        pl.BlockSpec((BN, H), lambda i: (i, 0)),
        pl.BlockSpec((BN, H), lambda i: (i, 0)),
        pl.BlockSpec((BN, H), lambda i: (i, 0)),
        pl.BlockSpec((BN, 16), lambda i: (i, 0)),
        pl.BlockSpec((1, H), lambda i: (0, 0)),
        pl.BlockSpec((H, H), lambda i: (0, 0)),
    ],
    out_specs=pl.BlockSpec((BN, H), lambda i: (i, 0)),
    out_shape=jax.ShapeDtypeStruct((N, H), jnp.float32),
)


def _tc_f_body(q0_ref, q1_ref, hs2_ref, dis_ref, b2_ref, batch_ref,
               wo_ref, bo_ref, out_ref, sum_ref, cnt_ref):
    g = pl.program_id(0)
    dis = dis_ref[...][:, 0:1]
    h2 = (q0_ref[...] + q1_ref[...] + hs2_ref[...]) * dis + b2_ref[...]
    bblk = batch_ref[0]
    gids = lax.broadcasted_iota(jnp.int32, (G, BN), 0)
    oh = (gids == bblk).astype(jnp.float32)
    psum = jnp.dot(oh, h2, preferred_element_type=jnp.float32)
    pcnt = jnp.broadcast_to(jnp.sum(oh, axis=1, keepdims=True), (G, H))

    @pl.when(g == 0)
    def _():
        sum_ref[...] = jnp.zeros_like(sum_ref)
        cnt_ref[...] = jnp.zeros_like(cnt_ref)

    sum_ref[...] += psum
    cnt_ref[...] += pcnt

    @pl.when(g == NBLK - 1)
    def _():
        pooled = sum_ref[...] / jnp.maximum(cnt_ref[...], 1.0)
        out_ref[...] = (
            jnp.dot(pooled, wo_ref[...], preferred_element_type=jnp.float32)
            + bo_ref[...]
        )


_tc_f = pl.pallas_call(
    _tc_f_body,
    grid=(NBLK,),
    in_specs=[
        pl.BlockSpec((BN, H), lambda i: (i, 0)),
        pl.BlockSpec((BN, H), lambda i: (i, 0)),
        pl.BlockSpec((BN, H), lambda i: (i, 0)),
        pl.BlockSpec((BN, 16), lambda i: (i, 0)),
        pl.BlockSpec((1, H), lambda i: (0, 0)),
        pl.BlockSpec((1, 1, BN), lambda i: (i, 0, 0)),
        pl.BlockSpec((H, 128), lambda i: (0, 0)),
        pl.BlockSpec((1, 128), lambda i: (0, 0)),
    ],
    out_specs=pl.BlockSpec((G, 128), lambda i: (0, 0)),
    out_shape=jax.ShapeDtypeStruct((G, 128), jnp.float32),
    scratch_shapes=[
        pltpu.VMEM((G, H), jnp.float32),
        pltpu.VMEM((G, H), jnp.float32),
    ],
)


def kernel(x, edge_index, batch, W1, b1, W2, b2, Wo, bo):
    src = edge_index[0].reshape(NW, CH, K)
    dst = edge_index[1].reshape(NW, CH, K)
    zeros_h = jnp.zeros((RPT, H), jnp.float32)
    zeros16 = jnp.zeros((RPT, 16), jnp.float32)
    ones16 = jnp.ones((K, 16), jnp.float32)

    degp = _sc_deg(dst, ones16, zeros16)
    dis16, hs1 = _tc_b(degp[0], degp[1], x, W1)
    p = _sc_scatter(src, dst, hs1, zeros_h)
    hs2 = _tc_d(p[0], p[1], hs1, dis16, b1.reshape(1, H), W2)
    q = _sc_scatter(src, dst, hs2, zeros_h)
    wo_pad = jnp.pad(Wo, ((0, 0), (0, 128 - C_OUT)))
    bo_pad = jnp.pad(bo, (0, 128 - C_OUT)).reshape(1, 128)
    outp = _tc_f(q[0], q[1], hs2, dis16, b2.reshape(1, H),
                 batch.reshape(NBLK, 1, BN), wo_pad, bo_pad)
    return outp[:, :C_OUT]


# speedup vs baseline: 29.5954x; 1.0620x over previous
import functools

import jax
import jax.numpy as jnp
from jax import lax
from jax.experimental import pallas as pl
from jax.experimental.pallas import tpu as pltpu
from jax.experimental.pallas import tpu_sc as plsc

N = 10000
E = 320000
D = 128
H = 128
G = 64
C_OUT = 19

NC = 2
NS = 16
NW = NC * NS
EPT = E // NW
K = 80
CH = EPT // K
RPT = N // NS
RCH = 125
BN = 1000
NBLK = N // BN


def _sc_scatter_body(src_hbm, dst_hbm, hs_hbm, zeros_hbm, out0_hbm, out1_hbm,
                     idx_s, idx_d, rows_a, rows_b, acc, sem_a, sem_b, sem_c):
    c = lax.axis_index("c")
    s = lax.axis_index("s")
    wid = c * NS + s
    row0 = s * RPT
    pltpu.async_copy(src_hbm.at[wid], idx_s, sem_a)
    pltpu.async_copy(dst_hbm.at[wid], idx_d, sem_b)
    pltpu.async_copy(zeros_hbm, acc.at[pl.ds(row0, RPT)], sem_c)
    pltpu.make_async_copy(src_hbm.at[wid], idx_s, sem_a).wait()
    pltpu.make_async_copy(dst_hbm.at[wid], idx_d, sem_b).wait()
    pltpu.make_async_copy(zeros_hbm, acc.at[pl.ds(row0, RPT)], sem_c).wait()
    plsc.subcore_barrier()

    pltpu.async_copy(hs_hbm.at[idx_s.at[0]], rows_a, sem_a)

    def body(j, carry):
        i0 = 2 * j
        pltpu.async_copy(hs_hbm.at[idx_s.at[i0 + 1]], rows_b, sem_b)
        pltpu.make_async_copy(hs_hbm.at[idx_s.at[i0]], rows_a, sem_a).wait()
        pltpu.sync_copy(rows_a, acc.at[idx_d.at[i0]], add=True)

        @pl.when(j < CH // 2 - 1)
        def _():
            pltpu.async_copy(hs_hbm.at[idx_s.at[i0 + 2]], rows_a, sem_a)

        pltpu.make_async_copy(hs_hbm.at[idx_s.at[i0 + 1]], rows_b,
                              sem_b).wait()
        pltpu.sync_copy(rows_b, acc.at[idx_d.at[i0 + 1]], add=True)
        return carry

    lax.fori_loop(0, CH // 2, body, 0)
    if CH % 2 == 1:
        pltpu.async_copy(hs_hbm.at[idx_s.at[CH - 1]], rows_a, sem_a).wait()
        pltpu.sync_copy(rows_a, acc.at[idx_d.at[CH - 1]], add=True)
    plsc.subcore_barrier()
    w0 = s * 624

    @pl.when(c == 0)
    def _():
        pltpu.sync_copy(acc.at[pl.ds(w0, 624)], out0_hbm.at[pl.ds(w0, 624)])

        @pl.when(s == NS - 1)
        def _():
            pltpu.sync_copy(acc.at[pl.ds(9984, 16)],
                            out0_hbm.at[pl.ds(9984, 16)])

    @pl.when(c == 1)
    def _():
        pltpu.sync_copy(acc.at[pl.ds(w0, 624)], out1_hbm.at[pl.ds(w0, 624)])

        @pl.when(s == NS - 1)
        def _():
            pltpu.sync_copy(acc.at[pl.ds(9984, 16)],
                            out1_hbm.at[pl.ds(9984, 16)])


_sc_scatter = functools.partial(
    pl.kernel,
    out_type=[jax.ShapeDtypeStruct((N, H), jnp.float32),
              jax.ShapeDtypeStruct((N, H), jnp.float32)],
    mesh=plsc.VectorSubcoreMesh(core_axis_name="c", subcore_axis_name="s"),
    scratch_types=[
        pltpu.VMEM((CH, K), jnp.int32),
        pltpu.VMEM((CH, K), jnp.int32),
        pltpu.VMEM((K, H), jnp.float32),
        pltpu.VMEM((K, H), jnp.float32),
        pltpu.VMEM_SHARED((N, H), jnp.float32),
        pltpu.SemaphoreType.DMA,
        pltpu.SemaphoreType.DMA,
        pltpu.SemaphoreType.DMA,
    ],
    compiler_params=pltpu.CompilerParams(use_tc_tiling_on_sc=False),
)(_sc_scatter_body)


def _sc_deg_body(dst_hbm, ones_hbm, zeros_hbm, out0_hbm, out1_hbm,
                 idx_d, onesb, acc, sem):
    c = lax.axis_index("c")
    s = lax.axis_index("s")
    wid = c * NS + s
    pltpu.sync_copy(dst_hbm.at[wid], idx_d)
    pltpu.sync_copy(ones_hbm, onesb)
    row0 = s * RPT
    pltpu.sync_copy(zeros_hbm, acc.at[pl.ds(row0, RPT)])
    plsc.subcore_barrier()

    def body(i, carry):
        pltpu.sync_copy(onesb, acc.at[idx_d.at[i]], add=True)
        return carry

    lax.fori_loop(0, CH, body, 0)
    plsc.subcore_barrier()
    w0 = s * 624

    @pl.when(c == 0)
    def _():
        pltpu.sync_copy(acc.at[pl.ds(w0, 624)], out0_hbm.at[pl.ds(w0, 624)])

        @pl.when(s == NS - 1)
        def _():
            pltpu.sync_copy(acc.at[pl.ds(9984, 16)],
                            out0_hbm.at[pl.ds(9984, 16)])

    @pl.when(c == 1)
    def _():
        pltpu.sync_copy(acc.at[pl.ds(w0, 624)], out1_hbm.at[pl.ds(w0, 624)])

        @pl.when(s == NS - 1)
        def _():
            pltpu.sync_copy(acc.at[pl.ds(9984, 16)],
                            out1_hbm.at[pl.ds(9984, 16)])


_sc_deg = functools.partial(
    pl.kernel,
    out_type=[jax.ShapeDtypeStruct((N, 16), jnp.float32),
              jax.ShapeDtypeStruct((N, 16), jnp.float32)],
    mesh=plsc.VectorSubcoreMesh(core_axis_name="c", subcore_axis_name="s"),
    scratch_types=[
        pltpu.VMEM((CH, K), jnp.int32),
        pltpu.VMEM((K, 16), jnp.float32),
        pltpu.VMEM_SHARED((N, 16), jnp.float32),
        pltpu.SemaphoreType.DMA,
    ],
    compiler_params=pltpu.CompilerParams(use_tc_tiling_on_sc=False),
)(_sc_deg_body)


def _tc_b_body(deg0_ref, deg1_ref, x_ref, w1_ref, dis_ref, hs_ref):
    deg = deg0_ref[...] + deg1_ref[...] + 1.0
    dis = lax.rsqrt(deg)
    dis_ref[...] = dis
    xw = jnp.dot(x_ref[...], w1_ref[...], preferred_element_type=jnp.float32)
    hs_ref[...] = xw * dis[:, 0:1]


_tc_b = pl.pallas_call(
    _tc_b_body,
    grid=(NBLK,),
    in_specs=[
        pl.BlockSpec((BN, 16), lambda i: (i, 0)),
        pl.BlockSpec((BN, 16), lambda i: (i, 0)),
        pl.BlockSpec((BN, D), lambda i: (i, 0)),
        pl.BlockSpec((D, H), lambda i: (0, 0)),
    ],
    out_specs=[
        pl.BlockSpec((BN, 16), lambda i: (i, 0)),
        pl.BlockSpec((BN, H), lambda i: (i, 0)),
    ],
    out_shape=[
        jax.ShapeDtypeStruct((N, 16), jnp.float32),
        jax.ShapeDtypeStruct((N, H), jnp.float32),
    ],
)


def _tc_d_body(p0_ref, p1_ref, hs1_ref, dis_ref, b1_ref, w2_ref, hs2_ref):
    dis = dis_ref[...][:, 0:1]
    acc = p0_ref[...] + p1_ref[...] + hs1_ref[...]
    h1 = jnp.maximum(acc * dis + b1_ref[...], 0.0)
    hw = jnp.dot(h1, w2_ref[...], preferred_element_type=jnp.float32)
    hs2_ref[...] = hw * dis


_tc_d = pl.pallas_call(
    _tc_d_body,
    grid=(NBLK,),
    in_specs=[
        pl.BlockSpec((BN, H), lambda i: (i, 0)),
        pl.BlockSpec((BN, H), lambda i: (i, 0)),
        pl.BlockSpec((BN, H), lambda i: (i, 0)),
        pl.BlockSpec((BN, 16), lambda i: (i, 0)),
        pl.BlockSpec((1, H), lambda i: (0, 0)),
        pl.BlockSpec((H, H), lambda i: (0, 0)),
    ],
    out_specs=pl.BlockSpec((BN, H), lambda i: (i, 0)),
    out_shape=jax.ShapeDtypeStruct((N, H), jnp.float32),
)


def _tc_f_body(q0_ref, q1_ref, hs2_ref, dis_ref, b2_ref, batch_ref,
               wo_ref, bo_ref, out_ref, sum_ref, cnt_ref):
    g = pl.program_id(0)
    dis = dis_ref[...][:, 0:1]
    h2 = (q0_ref[...] + q1_ref[...] + hs2_ref[...]) * dis + b2_ref[...]
    bblk = batch_ref[0]
    gids = lax.broadcasted_iota(jnp.int32, (G, BN), 0)
    oh = (gids == bblk).astype(jnp.float32)
    psum = jnp.dot(oh, h2, preferred_element_type=jnp.float32)
    pcnt = jnp.broadcast_to(jnp.sum(oh, axis=1, keepdims=True), (G, H))

    @pl.when(g == 0)
    def _():
        sum_ref[...] = jnp.zeros_like(sum_ref)
        cnt_ref[...] = jnp.zeros_like(cnt_ref)

    sum_ref[...] += psum
    cnt_ref[...] += pcnt

    @pl.when(g == NBLK - 1)
    def _():
        pooled = sum_ref[...] / jnp.maximum(cnt_ref[...], 1.0)
        out_ref[...] = (
            jnp.dot(pooled, wo_ref[...], preferred_element_type=jnp.float32)
            + bo_ref[...]
        )


_tc_f = pl.pallas_call(
    _tc_f_body,
    grid=(NBLK,),
    in_specs=[
        pl.BlockSpec((BN, H), lambda i: (i, 0)),
        pl.BlockSpec((BN, H), lambda i: (i, 0)),
        pl.BlockSpec((BN, H), lambda i: (i, 0)),
        pl.BlockSpec((BN, 16), lambda i: (i, 0)),
        pl.BlockSpec((1, H), lambda i: (0, 0)),
        pl.BlockSpec((1, 1, BN), lambda i: (i, 0, 0)),
        pl.BlockSpec((H, 128), lambda i: (0, 0)),
        pl.BlockSpec((1, 128), lambda i: (0, 0)),
    ],
    out_specs=pl.BlockSpec((G, 128), lambda i: (0, 0)),
    out_shape=jax.ShapeDtypeStruct((G, 128), jnp.float32),
    scratch_shapes=[
        pltpu.VMEM((G, H), jnp.float32),
        pltpu.VMEM((G, H), jnp.float32),
    ],
)


def kernel(x, edge_index, batch, W1, b1, W2, b2, Wo, bo):
    src = edge_index[0].reshape(NW, CH, K)
    dst = edge_index[1].reshape(NW, CH, K)
    zeros_h = jnp.zeros((RPT, H), jnp.float32)
    zeros16 = jnp.zeros((RPT, 16), jnp.float32)
    ones16 = jnp.ones((K, 16), jnp.float32)

    deg0, deg1 = _sc_deg(dst, ones16, zeros16)
    dis16, hs1 = _tc_b(deg0, deg1, x, W1)
    p0, p1 = _sc_scatter(src, dst, hs1, zeros_h)
    hs2 = _tc_d(p0, p1, hs1, dis16, b1.reshape(1, H), W2)
    q0, q1 = _sc_scatter(src, dst, hs2, zeros_h)
    wo_pad = jnp.pad(Wo, ((0, 0), (0, 128 - C_OUT)))
    bo_pad = jnp.pad(bo, (0, 128 - C_OUT)).reshape(1, 128)
    outp = _tc_f(q0, q1, hs2, dis16, b2.reshape(1, H),
                 batch.reshape(NBLK, 1, BN), wo_pad, bo_pad)
    return outp[:, :C_OUT]
